# baseline (device time: 27146 ns/iter reference)
import jax
import jax.numpy as jnp
from jax import lax
from jax.experimental import pallas as pl
from jax.experimental.pallas import tpu as pltpu

N_DEV = 4
B, SQ, SKV, HQ, DH = 2, 256, 256, 16, 64
H_PER = HQ // N_DEV
DM = 512


def kernel(x, Wq, K_ext, V_ext, Wo):
    my_i = lax.axis_index("i")
    h0 = my_i * H_PER
    K_sh = lax.dynamic_slice_in_dim(K_ext, h0, H_PER, axis=2)
    V_sh = lax.dynamic_slice_in_dim(V_ext, h0, H_PER, axis=2)
    K_sh = K_sh.transpose(0, 2, 1, 3).astype(jnp.bfloat16)
    V_sh = V_sh.transpose(0, 2, 1, 3).astype(jnp.bfloat16)
    xb = x.astype(jnp.bfloat16)
    Wqb = Wq.astype(jnp.bfloat16)
    Wob = Wo.astype(jnp.bfloat16)

    def body(x_ref, wq_ref, k_ref, v_ref, wo_ref, out_ref,
             comm_ref, send_sems, recv_sems):
        my_pos = lax.axis_index("i")
        p0 = jnp.bitwise_xor(my_pos, 1)
        p1 = jnp.bitwise_xor(my_pos, 2)

        barrier_sem = pltpu.get_barrier_semaphore()
        for p in (p0, p1):
            pl.semaphore_signal(barrier_sem, inc=1, device_id=(p,),
                                device_id_type=pl.DeviceIdType.MESH)
        pl.semaphore_wait(barrier_sem, 2)

        xf = x_ref[...].reshape(B * SQ, DM)
        q = jnp.dot(xf, wq_ref[...], preferred_element_type=jnp.float32)
        q = q.astype(jnp.bfloat16).reshape(B, SQ, H_PER, DH)

        qi = lax.broadcasted_iota(jnp.int32, (SQ, SKV), 0)
        ki = lax.broadcasted_iota(jnp.int32, (SQ, SKV), 1)
        mask = (jnp.abs(qi - ki) <= 128) | (ki < 32) | (qi < 32)

        for b in range(B):
            ctx_heads = []
            for h in range(H_PER):
                qb = q[b, :, h, :]
                kb = k_ref[b, h]
                vb = v_ref[b, h]
                s = lax.dot_general(
                    qb, kb, (((1,), (1,)), ((), ())),
                    preferred_element_type=jnp.float32,
                ) * 0.125
                s = jnp.where(mask, s, -1e9)
                s = s - s.max(axis=-1, keepdims=True)
                w = jnp.exp(s)
                w = w / w.sum(axis=-1, keepdims=True)
                ctx_heads.append(
                    jnp.dot(w.astype(jnp.bfloat16), vb,
                            preferred_element_type=jnp.float32)
                )
            ctx = jnp.concatenate(ctx_heads, axis=1).astype(jnp.bfloat16)
            part = jnp.dot(ctx, wo_ref[...],
                           preferred_element_type=jnp.float32)
            out_ref[b] = part
            comm_ref[0, b] = part.astype(jnp.bfloat16)

        rdma0 = pltpu.make_async_remote_copy(
            src_ref=comm_ref.at[0], dst_ref=comm_ref.at[1],
            send_sem=send_sems.at[0], recv_sem=recv_sems.at[0],
            device_id=(p0,), device_id_type=pl.DeviceIdType.MESH,
        )
        rdma0.start()
        rdma0.wait()
        acc = out_ref[...] + comm_ref[1].astype(jnp.float32)
        out_ref[...] = acc
        comm_ref[2] = acc.astype(jnp.bfloat16)

        rdma1 = pltpu.make_async_remote_copy(
            src_ref=comm_ref.at[2], dst_ref=comm_ref.at[3],
            send_sem=send_sems.at[1], recv_sem=recv_sems.at[1],
            device_id=(p1,), device_id_type=pl.DeviceIdType.MESH,
        )
        rdma1.start()
        rdma1.wait()
        out_ref[...] = out_ref[...] + comm_ref[3].astype(jnp.float32)

    out_shape = jax.ShapeDtypeStruct((B, SQ, DM), jnp.float32)
    return pl.pallas_call(
        body,
        out_shape=out_shape,
        in_specs=[pl.BlockSpec(memory_space=pltpu.VMEM)] * 5,
        out_specs=pl.BlockSpec(memory_space=pltpu.VMEM),
        scratch_shapes=[
            pltpu.VMEM((4, B, SQ, DM), jnp.bfloat16),
            pltpu.SemaphoreType.DMA((2,)),
            pltpu.SemaphoreType.DMA((2,)),
        ],
        compiler_params=pltpu.CompilerParams(collective_id=0),
    )(xb, Wqb, K_sh, V_sh, Wob)
